# baseline (speedup 1.0000x reference)
import functools

import jax
import jax.numpy as jnp
from jax import lax
from jax.experimental import pallas as pl
from jax.experimental.pallas import tpu as pltpu
from jax.experimental.pallas import tpu_sc as plsc

N = 10000
E = 320000
D = 128

NC = 2
NS = 16
W = 80

EDGES_PER_TILE = E // (NC * NS)
NWIN = EDGES_PER_TILE // W
N_PAD = 10240
ROWS_PER_TILE = N_PAD // NS
CNT_PER_TILE = N_PAD // NS

DEPTH = 4
NITER = NWIN // DEPTH

_mesh = plsc.VectorSubcoreMesh(core_axis_name="c", subcore_axis_name="s")


def _fill_idx(dst_buf, src_buf, off):
    for j in range(W // 16):
        dst_buf[pl.ds(j * 16, 16)] = src_buf[pl.ds(off + j * 16, 16)]


@functools.partial(
    pl.kernel,
    out_type=jax.ShapeDtypeStruct((NC, N_PAD, D), jnp.float32),
    mesh=_mesh,
    scratch_types=[
        pltpu.VMEM_SHARED((N_PAD, D), jnp.float32),
        [pltpu.VMEM((W,), jnp.int32) for _ in range(2 * DEPTH)],
        [pltpu.VMEM((W,), jnp.int32) for _ in range(2 * DEPTH)],
        [pltpu.VMEM((W, D), jnp.float32) for _ in range(DEPTH)],
        [pltpu.SemaphoreType.DMA for _ in range(2 * DEPTH)],
        [pltpu.SemaphoreType.DMA for _ in range(DEPTH)],
        [pltpu.SemaphoreType.DMA for _ in range(DEPTH)],
    ],
)
def _sc_agg(src_hbm, dst_hbm, x_hbm, zero_hbm, out_hbm,
            acc_sh, src_vs, dst_vs, rows_vs, isems, gsems, ssems):
    c = lax.axis_index("c")
    s = lax.axis_index("s")
    tid = c * NS + s
    base = tid * EDGES_PER_TILE
    NI = 2 * DEPTH

    pltpu.sync_copy(zero_hbm, acc_sh.at[pl.ds(s * ROWS_PER_TILE, ROWS_PER_TILE)])
    plsc.subcore_barrier()

    def load_idx(m, w):
        off = base + w * W
        pltpu.async_copy(src_hbm.at[pl.ds(off, W)], src_vs[m], isems[m])
        pltpu.async_copy(dst_hbm.at[pl.ds(off, W)], dst_vs[m], isems[m])

    def wait_idx(m):
        pltpu.make_async_copy(src_hbm.at[pl.ds(0, W)], src_vs[m],
                              isems[m]).wait()
        pltpu.make_async_copy(dst_hbm.at[pl.ds(0, W)], dst_vs[m],
                              isems[m]).wait()

    def gather(k, m):
        pltpu.async_copy(x_hbm.at[src_vs[m]], rows_vs[k], gsems[k])

    def wait_gather(k, m):
        pltpu.make_async_copy(x_hbm.at[src_vs[m]], rows_vs[k],
                              gsems[k]).wait()

    def scatter(k, m):
        pltpu.async_copy(rows_vs[k], acc_sh.at[dst_vs[m]], ssems[k], add=True)

    def wait_scatter(k, m):
        pltpu.make_async_copy(rows_vs[k], acc_sh.at[dst_vs[m]],
                              ssems[k]).wait()

    for m in range(NI):
        load_idx(m, m)
    for k in range(DEPTH):
        wait_idx(k)
        gather(k, k)

    def body(i, _):
        for j in range(NI):
            k = j % DEPTH
            wait_gather(k, j)
            scatter(k, j)
            wait_scatter(k, j)
            load_idx(j, (i + 1) * NI + j)
            m2 = (j + DEPTH) % NI
            wait_idx(m2)
            gather(k, m2)
        return _

    NROUND = NWIN // NI
    lax.fori_loop(0, NROUND - 1, body, None)

    for j in range(NI):
        k = j % DEPTH
        wait_gather(k, j)
        scatter(k, j)
        wait_scatter(k, j)
        if j < DEPTH:
            m2 = j + DEPTH
            wait_idx(m2)
            gather(k, m2)
    for w in range(NROUND * NI, NWIN):
        load_idx(0, w)
        wait_idx(0)
        gather(0, 0)
        wait_gather(0, 0)
        scatter(0, 0)
        wait_scatter(0, 0)

    plsc.subcore_barrier()
    pltpu.sync_copy(acc_sh.at[pl.ds(s * ROWS_PER_TILE, ROWS_PER_TILE)],
                    out_hbm.at[c, pl.ds(s * ROWS_PER_TILE, ROWS_PER_TILE)])


@functools.partial(
    pl.kernel,
    out_type=jax.ShapeDtypeStruct((NC, N_PAD), jnp.float32),
    mesh=_mesh,
    scratch_types=[
        pltpu.VMEM((EDGES_PER_TILE,), jnp.int32),
        pltpu.VMEM((W,), jnp.float32),
        pltpu.VMEM_SHARED((N_PAD,), jnp.float32),
        [pltpu.VMEM((W,), jnp.int32) for _ in range(DEPTH)],
        [pltpu.SemaphoreType.DMA for _ in range(DEPTH)],
    ],
)
def _sc_counts(dst_hbm, zero_hbm, out_hbm, dst_all, ones_v, cnt_sh,
               dst_vs, ssems):
    c = lax.axis_index("c")
    s = lax.axis_index("s")
    tid = c * NS + s
    base = tid * EDGES_PER_TILE

    for k in range(W // 16):
        ones_v[pl.ds(k * 16, 16)] = jnp.ones((16,), jnp.float32)

    pltpu.sync_copy(dst_hbm.at[pl.ds(base, EDGES_PER_TILE)], dst_all)
    pltpu.sync_copy(zero_hbm, cnt_sh.at[pl.ds(s * CNT_PER_TILE, CNT_PER_TILE)])
    plsc.subcore_barrier()

    for k in range(DEPTH):
        _fill_idx(dst_vs[k], dst_all, k * W)
        pltpu.async_copy(ones_v, cnt_sh.at[dst_vs[k]], ssems[k], add=True)

    def body(i, _):
        for k in range(DEPTH):
            w_next = (i + 1) * DEPTH + k
            pltpu.make_async_copy(ones_v, cnt_sh.at[dst_vs[k]],
                                  ssems[k]).wait()
            _fill_idx(dst_vs[k], dst_all, w_next * W)
            pltpu.async_copy(ones_v, cnt_sh.at[dst_vs[k]], ssems[k], add=True)
        return _

    lax.fori_loop(0, NITER - 1, body, None)
    for k in range(DEPTH):
        pltpu.make_async_copy(ones_v, cnt_sh.at[dst_vs[k]], ssems[k]).wait()
    for w in range(NITER * DEPTH, NWIN):
        _fill_idx(dst_vs[0], dst_all, w * W)
        pltpu.async_copy(ones_v, cnt_sh.at[dst_vs[0]], ssems[0], add=True)
        pltpu.make_async_copy(ones_v, cnt_sh.at[dst_vs[0]], ssems[0]).wait()
    plsc.subcore_barrier()

    pltpu.sync_copy(cnt_sh.at[pl.ds(s * CNT_PER_TILE, CNT_PER_TILE)],
                    out_hbm.at[c, pl.ds(s * CNT_PER_TILE, CNT_PER_TILE)])


F = D // 2
EPT = E // NS
NWIN2 = EPT // W
DEPTH2 = 4
NI2 = 2 * DEPTH2
NROUND2 = NWIN2 // NI2
N_PAD2 = 10112
STRIPE2 = N_PAD2 // NS
XROWS = N // 10


@functools.partial(
    pl.kernel,
    out_type=jax.ShapeDtypeStruct((NC, N_PAD2, F), jnp.float32),
    mesh=_mesh,
    scratch_types=[
        pltpu.VMEM_SHARED((N, F), jnp.float32),
        pltpu.VMEM_SHARED((N_PAD2, F), jnp.float32),
        pltpu.VMEM((NI2 * W,), jnp.int32),
        [pltpu.VMEM((W,), jnp.int32) for _ in range(NI2)],
        pltpu.VMEM((DEPTH2 * W, F), jnp.float32),
        [pltpu.SemaphoreType.DMA for _ in range(NI2)],
        [pltpu.SemaphoreType.DMA for _ in range(DEPTH2)],
        [pltpu.SemaphoreType.DMA for _ in range(DEPTH2)],
    ],
)
def _sc_agg2(src_hbm, dst_hbm, xh_hbm, zero_hbm, out_hbm,
             x_sh, acc_sh, src_ring, dst_vs, rows_ring, isems, gsems, ssems):
    c = lax.axis_index("c")
    s = lax.axis_index("s")
    base = s * EPT

    @pl.when(s < 10)
    def _():
        pltpu.sync_copy(xh_hbm.at[c, pl.ds(s * XROWS, XROWS)],
                        x_sh.at[pl.ds(s * XROWS, XROWS)])

    pltpu.sync_copy(zero_hbm, acc_sh.at[pl.ds(s * STRIPE2, STRIPE2)])
    plsc.subcore_barrier()

    def load_idx(m, w):
        off = base + w * W
        pltpu.async_copy(src_hbm.at[pl.ds(off, W)],
                         src_ring.at[pl.ds(m * W, W)], isems[m])
        pltpu.async_copy(dst_hbm.at[pl.ds(off, W)], dst_vs[m], isems[m])

    def wait_idx(m):
        pltpu.make_async_copy(src_hbm.at[pl.ds(0, W)],
                              src_ring.at[pl.ds(m * W, W)], isems[m]).wait()
        pltpu.make_async_copy(dst_hbm.at[pl.ds(0, W)], dst_vs[m],
                              isems[m]).wait()

    def gather(k, m):
        pltpu.async_copy(x_sh.at[src_ring.at[pl.ds(m * W, W)]],
                         rows_ring.at[pl.ds(k * W, W)], gsems[k])

    def wait_gather(k, m):
        pltpu.make_async_copy(x_sh.at[src_ring.at[pl.ds(m * W, W)]],
                              rows_ring.at[pl.ds(k * W, W)], gsems[k]).wait()

    def scatter(k, m):
        pltpu.async_copy(rows_ring.at[pl.ds(k * W, W)], acc_sh.at[dst_vs[m]],
                         ssems[k], add=True)

    def wait_scatter(k, m):
        pltpu.make_async_copy(rows_ring.at[pl.ds(k * W, W)],
                              acc_sh.at[dst_vs[m]], ssems[k]).wait()

    for m in range(NI2):
        load_idx(m, m)
    for k in range(DEPTH2):
        wait_idx(k)
        gather(k, k)

    def body(i, _):
        for j in range(NI2):
            k = j % DEPTH2
            wait_gather(k, j)
            scatter(k, j)
            wait_scatter(k, j)
            load_idx(j, (i + 1) * NI2 + j)
            m2 = (j + DEPTH2) % NI2
            wait_idx(m2)
            gather(k, m2)
        return _

    lax.fori_loop(0, NROUND2 - 1, body, None)

    for j in range(NI2):
        k = j % DEPTH2
        wait_gather(k, j)
        scatter(k, j)
        wait_scatter(k, j)
        if j < DEPTH2:
            m2 = j + DEPTH2
            wait_idx(m2)
            gather(k, m2)
    for w in range(NROUND2 * NI2, NWIN2):
        load_idx(0, w)
        wait_idx(0)
        gather(0, 0)
        wait_gather(0, 0)
        scatter(0, 0)
        wait_scatter(0, 0)

    plsc.subcore_barrier()
    pltpu.sync_copy(acc_sh.at[pl.ds(s * STRIPE2, STRIPE2)],
                    out_hbm.at[c, pl.ds(s * STRIPE2, STRIPE2)])


def _make_dense_body(full_out):
    def body(parts_ref, cnts_ref, xh_ref, wlt_ref, wrt_ref, bl_ref,
             g_ref, b_ref, o_ref):
        cnt = jnp.maximum(cnts_ref[0] + cnts_ref[1], 1.0)
        a = jnp.concatenate([parts_ref[0, :N, :], parts_ref[1, :N, :]],
                            axis=1)
        a = a * (1.0 / cnt)[:, None]
        xf = jnp.concatenate([xh_ref[0], xh_ref[1]], axis=1)
        y = (jnp.dot(a, wlt_ref[:], preferred_element_type=jnp.float32)
             + jnp.dot(xf, wrt_ref[:], preferred_element_type=jnp.float32)
             + bl_ref[:])
        mean = jnp.mean(y, axis=0, keepdims=True)
        var = jnp.mean((y - mean) ** 2, axis=0, keepdims=True)
        yn = (y - mean) * (lax.rsqrt(var + 1e-5) * g_ref[:]) + b_ref[:]
        yn = jnp.where(yn >= 0, yn, 0.01 * yn)
        if full_out:
            o_ref[:] = yn
        else:
            o_ref[0] = yn[:, :F]
            o_ref[1] = yn[:, F:]
    return body


_tc_dense_h = pl.pallas_call(
    _make_dense_body(False),
    out_shape=jax.ShapeDtypeStruct((NC, N, F), jnp.float32),
)
_tc_dense_full = pl.pallas_call(
    _make_dense_body(True),
    out_shape=jax.ShapeDtypeStruct((N, D), jnp.float32),
)


def kernel(x, edge_index, Wl0, bl0, Wr0, g0, b0, Wl1, bl1, Wr1, g1, b1,
           Wl2, bl2, Wr2, g2, b2, Wl3, bl3, Wr3, g3, b3):
    params = ((Wl0, bl0, Wr0, g0, b0), (Wl1, bl1, Wr1, g1, b1),
              (Wl2, bl2, Wr2, g2, b2), (Wl3, bl3, Wr3, g3, b3))
    src = edge_index[0].astype(jnp.int32)
    dst = edge_index[1].astype(jnp.int32)
    zero_rows = jnp.zeros((STRIPE2, F), jnp.float32)
    zero_cnt = jnp.zeros((CNT_PER_TILE,), jnp.float32)

    cnts = _sc_counts(dst, zero_cnt)[:, :N]
    xh = jnp.stack([x[:, :F], x[:, F:]])
    for i, (Wl, bl, Wr, g, b) in enumerate(params):
        parts = _sc_agg2(src, dst, xh, zero_rows)
        dense = _tc_dense_full if i == 3 else _tc_dense_h
        out = dense(parts, cnts, xh, Wl.T, Wr.T,
                    bl.reshape(1, D), g.reshape(1, D), b.reshape(1, D))
        xh = out
    return out


# speedup vs baseline: 1.1980x; 1.1980x over previous
import functools

import jax
import jax.numpy as jnp
from jax import lax
from jax.experimental import pallas as pl
from jax.experimental.pallas import tpu as pltpu
from jax.experimental.pallas import tpu_sc as plsc

N = 10000
E = 320000
D = 128

NC = 2
NS = 16
W = 80

EDGES_PER_TILE = E // (NC * NS)
NWIN = EDGES_PER_TILE // W
N_PAD = 10240
ROWS_PER_TILE = N_PAD // NS
CNT_PER_TILE = N_PAD // NS

DEPTH = 4
NITER = NWIN // DEPTH

_mesh = plsc.VectorSubcoreMesh(core_axis_name="c", subcore_axis_name="s")


def _fill_idx(dst_buf, src_buf, off):
    for j in range(W // 16):
        dst_buf[pl.ds(j * 16, 16)] = src_buf[pl.ds(off + j * 16, 16)]


@functools.partial(
    pl.kernel,
    out_type=jax.ShapeDtypeStruct((NC, N_PAD, D), jnp.float32),
    mesh=_mesh,
    scratch_types=[
        pltpu.VMEM_SHARED((N_PAD, D), jnp.float32),
        [pltpu.VMEM((W,), jnp.int32) for _ in range(2 * DEPTH)],
        [pltpu.VMEM((W,), jnp.int32) for _ in range(2 * DEPTH)],
        [pltpu.VMEM((W, D), jnp.float32) for _ in range(DEPTH)],
        [pltpu.SemaphoreType.DMA for _ in range(2 * DEPTH)],
        [pltpu.SemaphoreType.DMA for _ in range(DEPTH)],
        [pltpu.SemaphoreType.DMA for _ in range(DEPTH)],
    ],
)
def _sc_agg(src_hbm, dst_hbm, x_hbm, zero_hbm, out_hbm,
            acc_sh, src_vs, dst_vs, rows_vs, isems, gsems, ssems):
    c = lax.axis_index("c")
    s = lax.axis_index("s")
    tid = c * NS + s
    base = tid * EDGES_PER_TILE
    NI = 2 * DEPTH

    pltpu.sync_copy(zero_hbm, acc_sh.at[pl.ds(s * ROWS_PER_TILE, ROWS_PER_TILE)])
    plsc.subcore_barrier()

    def load_idx(m, w):
        off = base + w * W
        pltpu.async_copy(src_hbm.at[pl.ds(off, W)], src_vs[m], isems[m])
        pltpu.async_copy(dst_hbm.at[pl.ds(off, W)], dst_vs[m], isems[m])

    def wait_idx(m):
        pltpu.make_async_copy(src_hbm.at[pl.ds(0, W)], src_vs[m],
                              isems[m]).wait()
        pltpu.make_async_copy(dst_hbm.at[pl.ds(0, W)], dst_vs[m],
                              isems[m]).wait()

    def gather(k, m):
        pltpu.async_copy(x_hbm.at[src_vs[m]], rows_vs[k], gsems[k])

    def wait_gather(k, m):
        pltpu.make_async_copy(x_hbm.at[src_vs[m]], rows_vs[k],
                              gsems[k]).wait()

    def scatter(k, m):
        pltpu.async_copy(rows_vs[k], acc_sh.at[dst_vs[m]], ssems[k], add=True)

    def wait_scatter(k, m):
        pltpu.make_async_copy(rows_vs[k], acc_sh.at[dst_vs[m]],
                              ssems[k]).wait()

    def step(w, j, wait_s, load, gath):
        k = j % DEPTH
        m = j % NI
        wait_gather(k, m)
        scatter(k, m)
        if wait_s:
            wait_scatter((j - 2) % DEPTH, (j - 2) % NI)
        if load:
            load_idx((j - 2) % NI, w + 6)
        if gath:
            m2 = (j + 2) % NI
            wait_idx(m2)
            gather((j + 2) % DEPTH, m2)

    for m in range(6):
        load_idx(m, m)
    for k in range(2):
        wait_idx(k)
        gather(k, k)
    for w in range(NI):
        step(w, w, wait_s=(w >= 2), load=True, gath=True)

    def body(i, _):
        for j in range(NI):
            step(NI + i * NI + j, j, wait_s=True, load=True, gath=True)
        return _

    NSTEADY = (NWIN - 2 * NI - 5) // NI
    lax.fori_loop(0, NSTEADY, body, None)

    for w in range(NI + NSTEADY * NI, NWIN):
        step(w, w % NI, wait_s=True, load=(w + 6 <= NWIN - 1),
             gath=(w + 2 <= NWIN - 1))
    for w in (NWIN - 2, NWIN - 1):
        wait_scatter(w % DEPTH, w % NI)

    plsc.subcore_barrier()
    pltpu.sync_copy(acc_sh.at[pl.ds(s * ROWS_PER_TILE, ROWS_PER_TILE)],
                    out_hbm.at[c, pl.ds(s * ROWS_PER_TILE, ROWS_PER_TILE)])


@functools.partial(
    pl.kernel,
    out_type=jax.ShapeDtypeStruct((NC, N_PAD), jnp.float32),
    mesh=_mesh,
    scratch_types=[
        pltpu.VMEM((EDGES_PER_TILE,), jnp.int32),
        pltpu.VMEM((W,), jnp.float32),
        pltpu.VMEM_SHARED((N_PAD,), jnp.float32),
        [pltpu.VMEM((W,), jnp.int32) for _ in range(DEPTH)],
        [pltpu.SemaphoreType.DMA for _ in range(DEPTH)],
    ],
)
def _sc_counts(dst_hbm, zero_hbm, out_hbm, dst_all, ones_v, cnt_sh,
               dst_vs, ssems):
    c = lax.axis_index("c")
    s = lax.axis_index("s")
    tid = c * NS + s
    base = tid * EDGES_PER_TILE

    for k in range(W // 16):
        ones_v[pl.ds(k * 16, 16)] = jnp.ones((16,), jnp.float32)

    pltpu.sync_copy(dst_hbm.at[pl.ds(base, EDGES_PER_TILE)], dst_all)
    pltpu.sync_copy(zero_hbm, cnt_sh.at[pl.ds(s * CNT_PER_TILE, CNT_PER_TILE)])
    plsc.subcore_barrier()

    for k in range(DEPTH):
        _fill_idx(dst_vs[k], dst_all, k * W)
        pltpu.async_copy(ones_v, cnt_sh.at[dst_vs[k]], ssems[k], add=True)

    def body(i, _):
        for k in range(DEPTH):
            w_next = (i + 1) * DEPTH + k
            pltpu.make_async_copy(ones_v, cnt_sh.at[dst_vs[k]],
                                  ssems[k]).wait()
            _fill_idx(dst_vs[k], dst_all, w_next * W)
            pltpu.async_copy(ones_v, cnt_sh.at[dst_vs[k]], ssems[k], add=True)
        return _

    lax.fori_loop(0, NITER - 1, body, None)
    for k in range(DEPTH):
        pltpu.make_async_copy(ones_v, cnt_sh.at[dst_vs[k]], ssems[k]).wait()
    for w in range(NITER * DEPTH, NWIN):
        _fill_idx(dst_vs[0], dst_all, w * W)
        pltpu.async_copy(ones_v, cnt_sh.at[dst_vs[0]], ssems[0], add=True)
        pltpu.make_async_copy(ones_v, cnt_sh.at[dst_vs[0]], ssems[0]).wait()
    plsc.subcore_barrier()

    pltpu.sync_copy(cnt_sh.at[pl.ds(s * CNT_PER_TILE, CNT_PER_TILE)],
                    out_hbm.at[c, pl.ds(s * CNT_PER_TILE, CNT_PER_TILE)])


def _tc_dense_body(parts_ref, cnts_ref, x_ref, wlt_ref, wrt_ref, bl_ref,
                   g_ref, b_ref, o_ref):
    cnt = jnp.maximum(cnts_ref[0] + cnts_ref[1], 1.0)
    a = (parts_ref[0] + parts_ref[1]) * (1.0 / cnt)[:, None]
    y = (jnp.dot(a, wlt_ref[:], preferred_element_type=jnp.float32)
         + jnp.dot(x_ref[:], wrt_ref[:], preferred_element_type=jnp.float32)
         + bl_ref[:])
    mean = jnp.mean(y, axis=0, keepdims=True)
    var = jnp.mean((y - mean) ** 2, axis=0, keepdims=True)
    yn = (y - mean) * (lax.rsqrt(var + 1e-5) * g_ref[:]) + b_ref[:]
    o_ref[:] = jnp.where(yn >= 0, yn, 0.01 * yn)


_tc_dense = pl.pallas_call(
    _tc_dense_body,
    out_shape=jax.ShapeDtypeStruct((N, D), jnp.float32),
)


def kernel(x, edge_index, Wl0, bl0, Wr0, g0, b0, Wl1, bl1, Wr1, g1, b1,
           Wl2, bl2, Wr2, g2, b2, Wl3, bl3, Wr3, g3, b3):
    params = ((Wl0, bl0, Wr0, g0, b0), (Wl1, bl1, Wr1, g1, b1),
              (Wl2, bl2, Wr2, g2, b2), (Wl3, bl3, Wr3, g3, b3))
    src = edge_index[0].astype(jnp.int32)
    dst = edge_index[1].astype(jnp.int32)
    zero_rows = jnp.zeros((ROWS_PER_TILE, D), jnp.float32)
    zero_cnt = jnp.zeros((CNT_PER_TILE,), jnp.float32)

    cnts = _sc_counts(dst, zero_cnt)[:, :N]
    for Wl, bl, Wr, g, b in params:
        parts = _sc_agg(src, dst, x, zero_rows)[:, :N]
        x = _tc_dense(parts, cnts, x, Wl.T, Wr.T,
                      bl.reshape(1, D), g.reshape(1, D), b.reshape(1, D))
    return x


# speedup vs baseline: 1.2917x; 1.0783x over previous
import functools

import jax
import jax.numpy as jnp
from jax import lax
from jax.experimental import pallas as pl
from jax.experimental.pallas import tpu as pltpu
from jax.experimental.pallas import tpu_sc as plsc

N = 10000
E = 320000
D = 128
F = D // 2

NC = 2
NS = 16

W = 80
EDGES_PER_TILE = E // (NC * NS)
NWIN = EDGES_PER_TILE // W
N_PAD = 10240
CNT_PER_TILE = N_PAD // NS
DEPTH = 4
NITER = NWIN // DEPTH

AW = 128
EPT = E // NS
ANWIN = EPT // AW
TAILW = EPT - ANWIN * AW
ADEPTH = 6
ANI = 2 * ADEPTH
ANROUND = ANWIN // ANI
ZTILES = 10
ZROWS = N // ZTILES

_mesh = plsc.VectorSubcoreMesh(core_axis_name="c", subcore_axis_name="s")


def _fill_idx(dst_buf, src_buf, off):
    for j in range(W // 16):
        dst_buf[pl.ds(j * 16, 16)] = src_buf[pl.ds(off + j * 16, 16)]


@functools.partial(
    pl.kernel,
    out_type=jax.ShapeDtypeStruct((NC, N, F), jnp.float32),
    mesh=_mesh,
    scratch_types=[
        pltpu.VMEM_SHARED((N, F), jnp.float32),
        pltpu.VMEM((ANI * AW,), jnp.int32),
        [pltpu.VMEM((AW,), jnp.int32) for _ in range(ANI)],
        [pltpu.VMEM((AW, F), jnp.float32) for _ in range(ADEPTH)],
        pltpu.VMEM((TAILW,), jnp.int32),
        [pltpu.SemaphoreType.DMA for _ in range(ANI)],
        [pltpu.SemaphoreType.DMA for _ in range(ADEPTH)],
        [pltpu.SemaphoreType.DMA for _ in range(ADEPTH)],
    ],
    compiler_params=pltpu.CompilerParams(use_tc_tiling_on_sc=False),
)
def _sc_agg(src_hbm, dst_hbm, x0_hbm, x1_hbm, zero_hbm, out_hbm,
            acc_sh, src_ring, dst_vs, rows_vs, tail_dst, isems, gsems, ssems):
    c = lax.axis_index("c")
    s = lax.axis_index("s")
    base = s * EPT

    def _src(m, n=AW):
        return src_ring.at[pl.ds(m * AW, n)]

    @pl.when(s < ZTILES)
    def _():
        pltpu.sync_copy(zero_hbm, acc_sh.at[pl.ds(s * ZROWS, ZROWS)])

    plsc.subcore_barrier()

    def load_idx(m, w):
        off = base + w * AW
        pltpu.async_copy(src_hbm.at[pl.ds(off, AW)], _src(m), isems[m])
        pltpu.async_copy(dst_hbm.at[pl.ds(off, AW)], dst_vs[m], isems[m])

    def wait_idx(m):
        pltpu.make_async_copy(src_hbm.at[pl.ds(0, AW)], _src(m),
                              isems[m]).wait()
        pltpu.make_async_copy(dst_hbm.at[pl.ds(0, AW)], dst_vs[m],
                              isems[m]).wait()

    def gather(k, m):
        @pl.when(c == 0)
        def _():
            pltpu.async_copy(x0_hbm.at[_src(m)], rows_vs[k], gsems[k])

        @pl.when(c == 1)
        def _():
            pltpu.async_copy(x1_hbm.at[_src(m)], rows_vs[k], gsems[k])

    def wait_gather(k, m):
        pltpu.make_async_copy(x0_hbm.at[_src(m)], rows_vs[k],
                              gsems[k]).wait()

    def scatter(k, m):
        pltpu.async_copy(rows_vs[k], acc_sh.at[dst_vs[m]], ssems[k], add=True)

    def wait_scatter(k, m):
        pltpu.make_async_copy(rows_vs[k], acc_sh.at[dst_vs[m]],
                              ssems[k]).wait()

    for m in range(ANI):
        load_idx(m, m)
    for k in range(ADEPTH):
        wait_idx(k)
        gather(k, k)

    def body(i, _):
        for j in range(ANI):
            k = j % ADEPTH
            wait_gather(k, j)
            scatter(k, j)
            wait_scatter(k, j)
            load_idx(j, (i + 1) * ANI + j)
            m2 = (j + ADEPTH) % ANI
            wait_idx(m2)
            gather(k, m2)
        return _

    lax.fori_loop(0, ANROUND - 1, body, None)

    for j in range(ANI):
        k = j % ADEPTH
        wait_gather(k, j)
        scatter(k, j)
        wait_scatter(k, j)
        if j < ADEPTH:
            m2 = j + ADEPTH
            wait_idx(m2)
            gather(k, m2)

    toff = base + ANWIN * AW
    pltpu.async_copy(src_hbm.at[pl.ds(toff, TAILW)], _src(0, TAILW),
                     isems[0])
    pltpu.async_copy(dst_hbm.at[pl.ds(toff, TAILW)], tail_dst, isems[0])
    pltpu.make_async_copy(src_hbm.at[pl.ds(0, TAILW)], _src(0, TAILW),
                          isems[0]).wait()
    pltpu.make_async_copy(dst_hbm.at[pl.ds(0, TAILW)], tail_dst,
                          isems[0]).wait()

    @pl.when(c == 0)
    def _():
        pltpu.async_copy(x0_hbm.at[_src(0, TAILW)],
                         rows_vs[0].at[pl.ds(0, TAILW)], gsems[0])

    @pl.when(c == 1)
    def _():
        pltpu.async_copy(x1_hbm.at[_src(0, TAILW)],
                         rows_vs[0].at[pl.ds(0, TAILW)], gsems[0])

    pltpu.make_async_copy(x0_hbm.at[_src(0, TAILW)],
                          rows_vs[0].at[pl.ds(0, TAILW)], gsems[0]).wait()
    pltpu.async_copy(rows_vs[0].at[pl.ds(0, TAILW)], acc_sh.at[tail_dst],
                     ssems[0], add=True)
    pltpu.make_async_copy(rows_vs[0].at[pl.ds(0, TAILW)], acc_sh.at[tail_dst],
                          ssems[0]).wait()

    plsc.subcore_barrier()
    @pl.when(s < ZTILES)
    def _():
        pltpu.sync_copy(acc_sh.at[pl.ds(s * ZROWS, ZROWS)],
                        out_hbm.at[c, pl.ds(s * ZROWS, ZROWS)])


@functools.partial(
    pl.kernel,
    out_type=jax.ShapeDtypeStruct((NC, N_PAD), jnp.float32),
    mesh=_mesh,
    scratch_types=[
        pltpu.VMEM((EDGES_PER_TILE,), jnp.int32),
        pltpu.VMEM((W,), jnp.float32),
        pltpu.VMEM_SHARED((N_PAD,), jnp.float32),
        [pltpu.VMEM((W,), jnp.int32) for _ in range(DEPTH)],
        [pltpu.SemaphoreType.DMA for _ in range(DEPTH)],
    ],
)
def _sc_counts(dst_hbm, zero_hbm, out_hbm, dst_all, ones_v, cnt_sh,
               dst_vs, ssems):
    c = lax.axis_index("c")
    s = lax.axis_index("s")
    tid = c * NS + s
    base = tid * EDGES_PER_TILE

    for k in range(W // 16):
        ones_v[pl.ds(k * 16, 16)] = jnp.ones((16,), jnp.float32)

    pltpu.sync_copy(dst_hbm.at[pl.ds(base, EDGES_PER_TILE)], dst_all)
    pltpu.sync_copy(zero_hbm, cnt_sh.at[pl.ds(s * CNT_PER_TILE, CNT_PER_TILE)])
    plsc.subcore_barrier()

    for k in range(DEPTH):
        _fill_idx(dst_vs[k], dst_all, k * W)
        pltpu.async_copy(ones_v, cnt_sh.at[dst_vs[k]], ssems[k], add=True)

    def body(i, _):
        for k in range(DEPTH):
            w_next = (i + 1) * DEPTH + k
            pltpu.make_async_copy(ones_v, cnt_sh.at[dst_vs[k]],
                                  ssems[k]).wait()
            _fill_idx(dst_vs[k], dst_all, w_next * W)
            pltpu.async_copy(ones_v, cnt_sh.at[dst_vs[k]], ssems[k], add=True)
        return _

    lax.fori_loop(0, NITER - 1, body, None)
    for k in range(DEPTH):
        pltpu.make_async_copy(ones_v, cnt_sh.at[dst_vs[k]], ssems[k]).wait()
    for w in range(NITER * DEPTH, NWIN):
        _fill_idx(dst_vs[0], dst_all, w * W)
        pltpu.async_copy(ones_v, cnt_sh.at[dst_vs[0]], ssems[0], add=True)
        pltpu.make_async_copy(ones_v, cnt_sh.at[dst_vs[0]], ssems[0]).wait()
    plsc.subcore_barrier()

    pltpu.sync_copy(cnt_sh.at[pl.ds(s * CNT_PER_TILE, CNT_PER_TILE)],
                    out_hbm.at[c, pl.ds(s * CNT_PER_TILE, CNT_PER_TILE)])


def _make_dense_body(full_out):
    def body(parts_ref, cnts_ref, x0_ref, x1_ref, wlt_ref, wrt_ref, bl_ref,
             g_ref, b_ref, *outs):
        cnt = jnp.maximum(cnts_ref[0] + cnts_ref[1], 1.0)
        a = jnp.concatenate([parts_ref[0], parts_ref[1]], axis=1)
        a = a * (1.0 / cnt)[:, None]
        xf = jnp.concatenate([x0_ref[:], x1_ref[:]], axis=1)
        y = (jnp.dot(a, wlt_ref[:], preferred_element_type=jnp.float32)
             + jnp.dot(xf, wrt_ref[:], preferred_element_type=jnp.float32)
             + bl_ref[:])
        mean = jnp.mean(y, axis=0, keepdims=True)
        var = jnp.mean((y - mean) ** 2, axis=0, keepdims=True)
        yn = (y - mean) * (lax.rsqrt(var + 1e-5) * g_ref[:]) + b_ref[:]
        yn = jnp.where(yn >= 0, yn, 0.01 * yn)
        if full_out:
            outs[0][:] = yn
        else:
            outs[0][:] = yn[:, :F]
            outs[1][:] = yn[:, F:]
    return body


_tc_dense_h = pl.pallas_call(
    _make_dense_body(False),
    out_shape=(jax.ShapeDtypeStruct((N, F), jnp.float32),
               jax.ShapeDtypeStruct((N, F), jnp.float32)),
)
_tc_dense_full = pl.pallas_call(
    _make_dense_body(True),
    out_shape=jax.ShapeDtypeStruct((N, D), jnp.float32),
)


def kernel(x, edge_index, Wl0, bl0, Wr0, g0, b0, Wl1, bl1, Wr1, g1, b1,
           Wl2, bl2, Wr2, g2, b2, Wl3, bl3, Wr3, g3, b3):
    params = ((Wl0, bl0, Wr0, g0, b0), (Wl1, bl1, Wr1, g1, b1),
              (Wl2, bl2, Wr2, g2, b2), (Wl3, bl3, Wr3, g3, b3))
    src = edge_index[0].astype(jnp.int32)
    dst = edge_index[1].astype(jnp.int32)
    zero_rows = jnp.zeros((ZROWS, F), jnp.float32)
    zero_cnt = jnp.zeros((CNT_PER_TILE,), jnp.float32)

    cnts = _sc_counts(dst, zero_cnt)[:, :N]
    x0, x1 = x[:, :F], x[:, F:]
    for i, (Wl, bl, Wr, g, b) in enumerate(params):
        parts = _sc_agg(src, dst, x0, x1, zero_rows)
        dense = _tc_dense_full if i == 3 else _tc_dense_h
        out = dense(parts, cnts, x0, x1, Wl.T, Wr.T,
                    bl.reshape(1, D), g.reshape(1, D), b.reshape(1, D))
        if i < 3:
            x0, x1 = out
    return out


# speedup vs baseline: 1.4666x; 1.1354x over previous
import functools

import jax
import jax.numpy as jnp
from jax import lax
from jax.experimental import pallas as pl
from jax.experimental.pallas import tpu as pltpu
from jax.experimental.pallas import tpu_sc as plsc

N = 10000
E = 320000
D = 128

NC = 2
NS = 16
W = 88

EDGES_PER_TILE = E // (NC * NS)
NWIN = EDGES_PER_TILE // W
TAILW = EDGES_PER_TILE - NWIN * W
N_PAD = 10240
CNT_PER_TILE = N_PAD // NS
ZTILES = 10
ZROWS = N // ZTILES

DEPTH = 4

CW = 80
CNWIN = EDGES_PER_TILE // CW
NITER = CNWIN // DEPTH

_mesh = plsc.VectorSubcoreMesh(core_axis_name="c", subcore_axis_name="s")


def _fill_idx(dst_buf, src_buf, off):
    for j in range(CW // 16):
        dst_buf[pl.ds(j * 16, 16)] = src_buf[pl.ds(off + j * 16, 16)]


@functools.partial(
    pl.kernel,
    out_type=jax.ShapeDtypeStruct((NC, N, D), jnp.float32),
    mesh=_mesh,
    scratch_types=[
        pltpu.VMEM_SHARED((N, D), jnp.float32),
        [pltpu.VMEM((W,), jnp.int32) for _ in range(2 * DEPTH)],
        [pltpu.VMEM((W,), jnp.int32) for _ in range(2 * DEPTH)],
        [pltpu.VMEM((W, D), jnp.float32) for _ in range(DEPTH)],
        pltpu.VMEM((TAILW,), jnp.int32),
        [pltpu.SemaphoreType.DMA for _ in range(2 * DEPTH)],
        [pltpu.SemaphoreType.DMA for _ in range(DEPTH)],
        [pltpu.SemaphoreType.DMA for _ in range(DEPTH)],
    ],
)
def _sc_agg(src_hbm, dst_hbm, x_hbm, zero_hbm, out_hbm,
            acc_sh, src_vs, dst_vs, rows_vs, tail_dst, isems, gsems, ssems):
    c = lax.axis_index("c")
    s = lax.axis_index("s")
    tid = c * NS + s
    base = tid * EDGES_PER_TILE
    NI = 2 * DEPTH

    @pl.when(s < ZTILES)
    def _():
        pltpu.sync_copy(zero_hbm, acc_sh.at[pl.ds(s * ZROWS, ZROWS)])

    plsc.subcore_barrier()

    def load_idx(m, w):
        off = base + w * W
        pltpu.async_copy(src_hbm.at[pl.ds(off, W)], src_vs[m], isems[m])
        pltpu.async_copy(dst_hbm.at[pl.ds(off, W)], dst_vs[m], isems[m])

    def wait_idx(m):
        pltpu.make_async_copy(src_hbm.at[pl.ds(0, W)], src_vs[m],
                              isems[m]).wait()
        pltpu.make_async_copy(dst_hbm.at[pl.ds(0, W)], dst_vs[m],
                              isems[m]).wait()

    def gather(k, m):
        pltpu.async_copy(x_hbm.at[src_vs[m]], rows_vs[k], gsems[k])

    def wait_gather(k, m):
        pltpu.make_async_copy(x_hbm.at[src_vs[m]], rows_vs[k],
                              gsems[k]).wait()

    def scatter(k, m):
        pltpu.async_copy(rows_vs[k], acc_sh.at[dst_vs[m]], ssems[k], add=True)

    def wait_scatter(k, m):
        pltpu.make_async_copy(rows_vs[k], acc_sh.at[dst_vs[m]],
                              ssems[k]).wait()

    for m in range(NI):
        load_idx(m, m)
    for k in range(DEPTH):
        wait_idx(k)
        gather(k, k)

    def body(i, _):
        for j in range(NI):
            k = j % DEPTH
            wait_gather(k, j)
            scatter(k, j)
            wait_scatter(k, j)
            load_idx(j, (i + 1) * NI + j)
            m2 = (j + DEPTH) % NI
            wait_idx(m2)
            gather(k, m2)
        return _

    NROUND = NWIN // NI
    lax.fori_loop(0, NROUND - 1, body, None)

    for j in range(NI):
        k = j % DEPTH
        wait_gather(k, j)
        scatter(k, j)
        wait_scatter(k, j)
        if j < DEPTH:
            m2 = j + DEPTH
            wait_idx(m2)
            gather(k, m2)
    for w in range(NROUND * NI, NWIN):
        load_idx(0, w)
        wait_idx(0)
        gather(0, 0)
        wait_gather(0, 0)
        scatter(0, 0)
        wait_scatter(0, 0)

    toff = base + NWIN * W
    pltpu.async_copy(src_hbm.at[pl.ds(toff, TAILW)],
                     src_vs[0].at[pl.ds(0, TAILW)], isems[0])
    pltpu.async_copy(dst_hbm.at[pl.ds(toff, TAILW)], tail_dst, isems[0])
    pltpu.make_async_copy(src_hbm.at[pl.ds(0, TAILW)],
                          src_vs[0].at[pl.ds(0, TAILW)], isems[0]).wait()
    pltpu.make_async_copy(dst_hbm.at[pl.ds(0, TAILW)], tail_dst,
                          isems[0]).wait()
    pltpu.async_copy(x_hbm.at[src_vs[0].at[pl.ds(0, TAILW)]],
                     rows_vs[0].at[pl.ds(0, TAILW)], gsems[0])
    pltpu.make_async_copy(x_hbm.at[src_vs[0].at[pl.ds(0, TAILW)]],
                          rows_vs[0].at[pl.ds(0, TAILW)], gsems[0]).wait()
    pltpu.async_copy(rows_vs[0].at[pl.ds(0, TAILW)], acc_sh.at[tail_dst],
                     ssems[0], add=True)
    pltpu.make_async_copy(rows_vs[0].at[pl.ds(0, TAILW)], acc_sh.at[tail_dst],
                          ssems[0]).wait()

    plsc.subcore_barrier()
    @pl.when(s < ZTILES)
    def _():
        pltpu.sync_copy(acc_sh.at[pl.ds(s * ZROWS, ZROWS)],
                        out_hbm.at[c, pl.ds(s * ZROWS, ZROWS)])


@functools.partial(
    pl.kernel,
    out_type=jax.ShapeDtypeStruct((NC, N_PAD), jnp.float32),
    mesh=_mesh,
    scratch_types=[
        pltpu.VMEM((EDGES_PER_TILE,), jnp.int32),
        pltpu.VMEM((CW,), jnp.float32),
        pltpu.VMEM_SHARED((N_PAD,), jnp.float32),
        [pltpu.VMEM((CW,), jnp.int32) for _ in range(DEPTH)],
        [pltpu.SemaphoreType.DMA for _ in range(DEPTH)],
    ],
)
def _sc_counts(dst_hbm, zero_hbm, out_hbm, dst_all, ones_v, cnt_sh,
               dst_vs, ssems):
    c = lax.axis_index("c")
    s = lax.axis_index("s")
    tid = c * NS + s
    base = tid * EDGES_PER_TILE

    for k in range(CW // 16):
        ones_v[pl.ds(k * 16, 16)] = jnp.ones((16,), jnp.float32)

    pltpu.sync_copy(dst_hbm.at[pl.ds(base, EDGES_PER_TILE)], dst_all)
    pltpu.sync_copy(zero_hbm, cnt_sh.at[pl.ds(s * CNT_PER_TILE, CNT_PER_TILE)])
    plsc.subcore_barrier()

    for k in range(DEPTH):
        _fill_idx(dst_vs[k], dst_all, k * CW)
        pltpu.async_copy(ones_v, cnt_sh.at[dst_vs[k]], ssems[k], add=True)

    def body(i, _):
        for k in range(DEPTH):
            w_next = (i + 1) * DEPTH + k
            pltpu.make_async_copy(ones_v, cnt_sh.at[dst_vs[k]],
                                  ssems[k]).wait()
            _fill_idx(dst_vs[k], dst_all, w_next * CW)
            pltpu.async_copy(ones_v, cnt_sh.at[dst_vs[k]], ssems[k], add=True)
        return _

    lax.fori_loop(0, NITER - 1, body, None)
    for k in range(DEPTH):
        pltpu.make_async_copy(ones_v, cnt_sh.at[dst_vs[k]], ssems[k]).wait()
    for w in range(NITER * DEPTH, CNWIN):
        _fill_idx(dst_vs[0], dst_all, w * CW)
        pltpu.async_copy(ones_v, cnt_sh.at[dst_vs[0]], ssems[0], add=True)
        pltpu.make_async_copy(ones_v, cnt_sh.at[dst_vs[0]], ssems[0]).wait()
    plsc.subcore_barrier()

    pltpu.sync_copy(cnt_sh.at[pl.ds(s * CNT_PER_TILE, CNT_PER_TILE)],
                    out_hbm.at[c, pl.ds(s * CNT_PER_TILE, CNT_PER_TILE)])


def _tc_dense_body(parts_ref, cnts_ref, x_ref, wlt_ref, wrt_ref, bl_ref,
                   g_ref, b_ref, o_ref):
    cnt = jnp.maximum(cnts_ref[0] + cnts_ref[1], 1.0)
    a = (parts_ref[0] + parts_ref[1]) * (1.0 / cnt)[:, None]
    y = (jnp.dot(a, wlt_ref[:], preferred_element_type=jnp.float32)
         + jnp.dot(x_ref[:], wrt_ref[:], preferred_element_type=jnp.float32)
         + bl_ref[:])
    mean = jnp.mean(y, axis=0, keepdims=True)
    var = jnp.mean((y - mean) ** 2, axis=0, keepdims=True)
    yn = (y - mean) * (lax.rsqrt(var + 1e-5) * g_ref[:]) + b_ref[:]
    o_ref[:] = jnp.where(yn >= 0, yn, 0.01 * yn)


_tc_dense = pl.pallas_call(
    _tc_dense_body,
    out_shape=jax.ShapeDtypeStruct((N, D), jnp.float32),
)


def kernel(x, edge_index, Wl0, bl0, Wr0, g0, b0, Wl1, bl1, Wr1, g1, b1,
           Wl2, bl2, Wr2, g2, b2, Wl3, bl3, Wr3, g3, b3):
    params = ((Wl0, bl0, Wr0, g0, b0), (Wl1, bl1, Wr1, g1, b1),
              (Wl2, bl2, Wr2, g2, b2), (Wl3, bl3, Wr3, g3, b3))
    src = edge_index[0].astype(jnp.int32)
    dst = edge_index[1].astype(jnp.int32)
    zero_rows = jnp.zeros((ZROWS, D), jnp.float32)
    zero_cnt = jnp.zeros((CNT_PER_TILE,), jnp.float32)

    cnts = _sc_counts(dst, zero_cnt)[:, :N]
    for Wl, bl, Wr, g, b in params:
        parts = _sc_agg(src, dst, x, zero_rows)
        x = _tc_dense(parts, cnts, x, Wl.T, Wr.T,
                      bl.reshape(1, D), g.reshape(1, D), b.reshape(1, D))
    return x


# speedup vs baseline: 1.4684x; 1.0012x over previous
import functools

import jax
import jax.numpy as jnp
from jax import lax
from jax.experimental import pallas as pl
from jax.experimental.pallas import tpu as pltpu
from jax.experimental.pallas import tpu_sc as plsc

N = 10000
E = 320000
D = 128

NC = 2
NS = 16
W = 88

EDGES_PER_TILE = E // (NC * NS)
NWIN = EDGES_PER_TILE // W
TAILW = EDGES_PER_TILE - NWIN * W
N_PAD = 10240
CNT_PER_TILE = N_PAD // NS
ZTILES = 10
ZROWS = N // ZTILES

DEPTH = 4

CW = 80
CNWIN = EDGES_PER_TILE // CW
NITER = CNWIN // DEPTH

_mesh = plsc.VectorSubcoreMesh(core_axis_name="c", subcore_axis_name="s")


def _fill_idx(dst_buf, src_buf, off):
    for j in range(CW // 16):
        dst_buf[pl.ds(j * 16, 16)] = src_buf[pl.ds(off + j * 16, 16)]


@functools.partial(
    pl.kernel,
    out_type=jax.ShapeDtypeStruct((NC, N, D), jnp.float32),
    mesh=_mesh,
    scratch_types=[
        pltpu.VMEM_SHARED((N, D), jnp.float32),
        [pltpu.VMEM((W,), jnp.int32) for _ in range(2 * DEPTH)],
        [pltpu.VMEM((W,), jnp.int32) for _ in range(2 * DEPTH)],
        [pltpu.VMEM((W, D), jnp.float32) for _ in range(DEPTH)],
        pltpu.VMEM((TAILW,), jnp.int32),
        [pltpu.SemaphoreType.DMA for _ in range(2 * DEPTH)],
        [pltpu.SemaphoreType.DMA for _ in range(DEPTH)],
        [pltpu.SemaphoreType.DMA for _ in range(DEPTH)],
    ],
    compiler_params=pltpu.CompilerParams(skip_device_barrier=True),
)
def _sc_agg(src_hbm, dst_hbm, x_hbm, zero_hbm, out_hbm,
            acc_sh, src_vs, dst_vs, rows_vs, tail_dst, isems, gsems, ssems):
    c = lax.axis_index("c")
    s = lax.axis_index("s")
    tid = c * NS + s
    base = tid * EDGES_PER_TILE
    NI = 2 * DEPTH

    @pl.when(s < ZTILES)
    def _():
        pltpu.sync_copy(zero_hbm, acc_sh.at[pl.ds(s * ZROWS, ZROWS)])

    plsc.subcore_barrier()

    def load_idx(m, w):
        off = base + w * W
        pltpu.async_copy(src_hbm.at[pl.ds(off, W)], src_vs[m], isems[m])
        pltpu.async_copy(dst_hbm.at[pl.ds(off, W)], dst_vs[m], isems[m])

    def wait_idx(m):
        pltpu.make_async_copy(src_hbm.at[pl.ds(0, W)], src_vs[m],
                              isems[m]).wait()
        pltpu.make_async_copy(dst_hbm.at[pl.ds(0, W)], dst_vs[m],
                              isems[m]).wait()

    def gather(k, m):
        pltpu.async_copy(x_hbm.at[src_vs[m]], rows_vs[k], gsems[k])

    def wait_gather(k, m):
        pltpu.make_async_copy(x_hbm.at[src_vs[m]], rows_vs[k],
                              gsems[k]).wait()

    def scatter(k, m):
        pltpu.async_copy(rows_vs[k], acc_sh.at[dst_vs[m]], ssems[k], add=True)

    def wait_scatter(k, m):
        pltpu.make_async_copy(rows_vs[k], acc_sh.at[dst_vs[m]],
                              ssems[k]).wait()

    for m in range(NI):
        load_idx(m, m)
    for k in range(DEPTH):
        wait_idx(k)
        gather(k, k)

    def body(i, _):
        for j in range(NI):
            k = j % DEPTH
            wait_gather(k, j)
            scatter(k, j)
            wait_scatter(k, j)
            load_idx(j, (i + 1) * NI + j)
            m2 = (j + DEPTH) % NI
            wait_idx(m2)
            gather(k, m2)
        return _

    NROUND = NWIN // NI
    lax.fori_loop(0, NROUND - 1, body, None)

    for j in range(NI):
        k = j % DEPTH
        wait_gather(k, j)
        scatter(k, j)
        wait_scatter(k, j)
        if j < DEPTH:
            m2 = j + DEPTH
            wait_idx(m2)
            gather(k, m2)
    for w in range(NROUND * NI, NWIN):
        load_idx(0, w)
        wait_idx(0)
        gather(0, 0)
        wait_gather(0, 0)
        scatter(0, 0)
        wait_scatter(0, 0)

    toff = base + NWIN * W
    pltpu.async_copy(src_hbm.at[pl.ds(toff, TAILW)],
                     src_vs[0].at[pl.ds(0, TAILW)], isems[0])
    pltpu.async_copy(dst_hbm.at[pl.ds(toff, TAILW)], tail_dst, isems[0])
    pltpu.make_async_copy(src_hbm.at[pl.ds(0, TAILW)],
                          src_vs[0].at[pl.ds(0, TAILW)], isems[0]).wait()
    pltpu.make_async_copy(dst_hbm.at[pl.ds(0, TAILW)], tail_dst,
                          isems[0]).wait()
    pltpu.async_copy(x_hbm.at[src_vs[0].at[pl.ds(0, TAILW)]],
                     rows_vs[0].at[pl.ds(0, TAILW)], gsems[0])
    pltpu.make_async_copy(x_hbm.at[src_vs[0].at[pl.ds(0, TAILW)]],
                          rows_vs[0].at[pl.ds(0, TAILW)], gsems[0]).wait()
    pltpu.async_copy(rows_vs[0].at[pl.ds(0, TAILW)], acc_sh.at[tail_dst],
                     ssems[0], add=True)
    pltpu.make_async_copy(rows_vs[0].at[pl.ds(0, TAILW)], acc_sh.at[tail_dst],
                          ssems[0]).wait()

    plsc.subcore_barrier()
    @pl.when(s < ZTILES)
    def _():
        pltpu.sync_copy(acc_sh.at[pl.ds(s * ZROWS, ZROWS)],
                        out_hbm.at[c, pl.ds(s * ZROWS, ZROWS)])


@functools.partial(
    pl.kernel,
    out_type=jax.ShapeDtypeStruct((NC, N_PAD), jnp.float32),
    mesh=_mesh,
    scratch_types=[
        pltpu.VMEM((EDGES_PER_TILE,), jnp.int32),
        pltpu.VMEM((CW,), jnp.float32),
        pltpu.VMEM_SHARED((N_PAD,), jnp.float32),
        [pltpu.VMEM((CW,), jnp.int32) for _ in range(DEPTH)],
        [pltpu.SemaphoreType.DMA for _ in range(DEPTH)],
    ],
    compiler_params=pltpu.CompilerParams(skip_device_barrier=True),
)
def _sc_counts(dst_hbm, zero_hbm, out_hbm, dst_all, ones_v, cnt_sh,
               dst_vs, ssems):
    c = lax.axis_index("c")
    s = lax.axis_index("s")
    tid = c * NS + s
    base = tid * EDGES_PER_TILE

    for k in range(CW // 16):
        ones_v[pl.ds(k * 16, 16)] = jnp.ones((16,), jnp.float32)

    pltpu.sync_copy(dst_hbm.at[pl.ds(base, EDGES_PER_TILE)], dst_all)
    pltpu.sync_copy(zero_hbm, cnt_sh.at[pl.ds(s * CNT_PER_TILE, CNT_PER_TILE)])
    plsc.subcore_barrier()

    for k in range(DEPTH):
        _fill_idx(dst_vs[k], dst_all, k * CW)
        pltpu.async_copy(ones_v, cnt_sh.at[dst_vs[k]], ssems[k], add=True)

    def body(i, _):
        for k in range(DEPTH):
            w_next = (i + 1) * DEPTH + k
            pltpu.make_async_copy(ones_v, cnt_sh.at[dst_vs[k]],
                                  ssems[k]).wait()
            _fill_idx(dst_vs[k], dst_all, w_next * CW)
            pltpu.async_copy(ones_v, cnt_sh.at[dst_vs[k]], ssems[k], add=True)
        return _

    lax.fori_loop(0, NITER - 1, body, None)
    for k in range(DEPTH):
        pltpu.make_async_copy(ones_v, cnt_sh.at[dst_vs[k]], ssems[k]).wait()
    for w in range(NITER * DEPTH, CNWIN):
        _fill_idx(dst_vs[0], dst_all, w * CW)
        pltpu.async_copy(ones_v, cnt_sh.at[dst_vs[0]], ssems[0], add=True)
        pltpu.make_async_copy(ones_v, cnt_sh.at[dst_vs[0]], ssems[0]).wait()
    plsc.subcore_barrier()

    pltpu.sync_copy(cnt_sh.at[pl.ds(s * CNT_PER_TILE, CNT_PER_TILE)],
                    out_hbm.at[c, pl.ds(s * CNT_PER_TILE, CNT_PER_TILE)])


def _tc_dense_body(parts_ref, cnts_ref, x_ref, wlt_ref, wrt_ref, bl_ref,
                   g_ref, b_ref, o_ref):
    cnt = jnp.maximum(cnts_ref[0] + cnts_ref[1], 1.0)
    a = (parts_ref[0] + parts_ref[1]) * (1.0 / cnt)[:, None]
    y = (jnp.dot(a, wlt_ref[:], preferred_element_type=jnp.float32)
         + jnp.dot(x_ref[:], wrt_ref[:], preferred_element_type=jnp.float32)
         + bl_ref[:])
    mean = jnp.mean(y, axis=0, keepdims=True)
    var = jnp.mean((y - mean) ** 2, axis=0, keepdims=True)
    yn = (y - mean) * (lax.rsqrt(var + 1e-5) * g_ref[:]) + b_ref[:]
    o_ref[:] = jnp.where(yn >= 0, yn, 0.01 * yn)


_tc_dense = pl.pallas_call(
    _tc_dense_body,
    out_shape=jax.ShapeDtypeStruct((N, D), jnp.float32),
)


def kernel(x, edge_index, Wl0, bl0, Wr0, g0, b0, Wl1, bl1, Wr1, g1, b1,
           Wl2, bl2, Wr2, g2, b2, Wl3, bl3, Wr3, g3, b3):
    params = ((Wl0, bl0, Wr0, g0, b0), (Wl1, bl1, Wr1, g1, b1),
              (Wl2, bl2, Wr2, g2, b2), (Wl3, bl3, Wr3, g3, b3))
    src = edge_index[0].astype(jnp.int32)
    dst = edge_index[1].astype(jnp.int32)
    zero_rows = jnp.zeros((ZROWS, D), jnp.float32)
    zero_cnt = jnp.zeros((CNT_PER_TILE,), jnp.float32)

    cnts = _sc_counts(dst, zero_cnt)[:, :N]
    for Wl, bl, Wr, g, b in params:
        parts = _sc_agg(src, dst, x, zero_rows)
        x = _tc_dense(parts, cnts, x, Wl.T, Wr.T,
                      bl.reshape(1, D), g.reshape(1, D), b.reshape(1, D))
    return x
